# baseline (speedup 1.0000x reference)
import functools
import math

import jax
from jax import lax
import jax.numpy as jnp
from jax.experimental import pallas as pl
from jax.experimental.pallas import tpu as pltpu
from jax.experimental.pallas import tpu_sc as plsc

_SMOOTHING = 0.1
_CONF = 1.0 - _SMOOTHING
_BLK_R = 16
_SC_ROWS = 256


def _dense_kernel(a_ref, b_ref, strip_ref, out_ref):
    @pl.when(pl.program_id(0) == 0)
    def _init():
        out_ref[0, 0] = jnp.sum(strip_ref[...])

    out_ref[0, 0] += jnp.sum(a_ref[...]) + jnp.sum(b_ref[...])


_SC_CHUNK = 3200


def _sc_dense_body(pred_hbm, out_hbm, bufa, bufb, buft, acc_v, sem,
                   *, r0, bands_per_tec, n_full, tail_w, NC):
    wid = lax.axis_index("s") * NC + lax.axis_index("c")
    acc = jnp.zeros((16,), jnp.float32)
    for band in range(bands_per_tec):
        row0 = r0 + 8 * (wid * bands_per_tec + band)
        rows8 = pl.ds(row0, 8)
        n_chunks = n_full + 1
        bufs = [bufa, bufb]
        cps = [None] * n_chunks

        def _issue(k):
            if k < n_full:
                return pltpu.async_copy(
                    pred_hbm.at[rows8, pl.ds(k * _SC_CHUNK, _SC_CHUNK)],
                    bufs[k % 2], sem)
            return pltpu.async_copy(
                pred_hbm.at[rows8, pl.ds(n_full * _SC_CHUNK, tail_w)],
                buft, sem)

        cps[0] = _issue(0)
        if n_chunks > 1:
            cps[1] = _issue(1)
        for k in range(n_chunks):
            cps[k].wait()
            buf = bufs[k % 2] if k < n_full else buft
            w = _SC_CHUNK if k < n_full else tail_w
            for r in range(8):
                acc = lax.fori_loop(
                    0, w // 16,
                    lambda i, a, _buf=buf, _r=r: a + _buf[_r, pl.ds(i * 16, 16)],
                    acc)
            if k + 2 < n_chunks:
                cps[k + 2] = _issue(k + 2)
    acc_v[...] = acc
    pltpu.sync_copy(acc_v, out_hbm.at[wid])


def _sc_gather_body(pred_hbm, gold_hbm, out_hbm, gold_v, win_v, p0win_v,
                    acc_v, sem, *, V, b_per_w, n_sub, NC):
    s = _SMOOTHING / (V - 2)
    slogs = s * math.log(s)
    wid = lax.axis_index("s") * NC + lax.axis_index("c")
    base = wid * b_per_w
    pltpu.sync_copy(gold_hbm.at[pl.ds(base, b_per_w)], gold_v)
    iota16 = lax.iota(jnp.int32, 16)
    copies = []
    for j in range(n_sub):
        g = gold_v[pl.ds(j * 16, 16)]
        col0_vec = lax.bitwise_and(g, ~127)
        for i in range(16):
            r = j * 16 + i
            col0 = pl.multiple_of(col0_vec[i], 128)
            copies.append(pltpu.async_copy(
                pred_hbm.at[pl.ds(base + (r & ~7), 8),
                            pl.ds(col0, 128)],
                win_v.at[r], sem))
    for t in range(b_per_w // 8):
        copies.append(pltpu.async_copy(
            pred_hbm.at[pl.ds(base + 8 * t, 8), pl.ds(0, 128)],
            p0win_v.at[t], sem))
    for cp in copies:
        cp.wait()
    acc = jnp.zeros((16,), jnp.float32)
    mask0 = jnp.where(iota16 == 0, 1.0, 0.0).astype(jnp.float32)
    for j in range(n_sub):
        g = gold_v[pl.ds(j * 16, 16)]
        lane_vec = lax.bitwise_and(g, 127)
        for i in range(16):
            r = j * 16 + i
            lane = lane_vec[i]
            for k in range(8):
                pgv = jnp.where(iota16 + 16 * k == lane,
                                win_v[r, r & 7, pl.ds(16 * k, 16)], 0.0)
                acc = acc + (s - _CONF) * pgv
            p0v = p0win_v[r // 8, r & 7, pl.ds(0, 16)] * mask0
            gzf = jnp.where(g[i] == 0, 1.0, 0.0).astype(jnp.float32)
            acc = acc + s * p0v + gzf * (slogs * mask0 - s * p0v)
    acc_v[...] = acc
    pltpu.sync_copy(acc_v, out_hbm.at[wid])


def kernel(pred, gold):
    B, V = pred.shape
    s = _SMOOTHING / (V - 2)
    k1 = (V - 2) * s * math.log(s) + _CONF * math.log(_CONF)

    info = plsc.get_sparse_core_info()
    NC, NS = info.num_cores, info.num_subcores
    NW = NC * NS
    b_per_w = B // NW
    n_sub = b_per_w // 16

    sc_rows = _SC_ROWS
    r0 = B - sc_rows
    v128 = (V // 128) * 128
    strip = lax.slice(pred, (r0, v128), (B, V))

    blk_r = _BLK_R
    n_steps = r0 // (2 * blk_r)
    dense = pl.pallas_call(
        _dense_kernel,
        grid=(n_steps,),
        in_specs=[
            pl.BlockSpec((blk_r, V), lambda i: (2 * i, 0)),
            pl.BlockSpec((blk_r, V), lambda i: (2 * i + 1, 0)),
            pl.BlockSpec((sc_rows, V - v128), lambda i: (0, 0)),
        ],
        out_specs=pl.BlockSpec(memory_space=pltpu.SMEM),
        out_shape=jax.ShapeDtypeStruct((1, 1), jnp.float32),
        compiler_params=pltpu.CompilerParams(
            dimension_semantics=("arbitrary",),
        ),
    )(pred, pred, strip)

    n_full = v128 // _SC_CHUNK
    tail_w = v128 - n_full * _SC_CHUNK
    bands_per_tec = sc_rows // (8 * NW)
    sc_dense_fn = functools.partial(
        pl.kernel,
        mesh=plsc.VectorSubcoreMesh(core_axis_name="c", subcore_axis_name="s"),
        out_type=jax.ShapeDtypeStruct((NW, 16), jnp.float32),
        scratch_types=[
            pltpu.VMEM((8, _SC_CHUNK), jnp.float32),
            pltpu.VMEM((8, _SC_CHUNK), jnp.float32),
            pltpu.VMEM((8, tail_w), jnp.float32),
            pltpu.VMEM((16,), jnp.float32),
            pltpu.SemaphoreType.DMA,
        ],
    )(functools.partial(_sc_dense_body, r0=r0, bands_per_tec=bands_per_tec,
                        n_full=n_full, tail_w=tail_w, NC=NC))
    sc_dense = sc_dense_fn(pred)
    sc_fn = functools.partial(
        pl.kernel,
        mesh=plsc.VectorSubcoreMesh(core_axis_name="c", subcore_axis_name="s"),
        out_type=jax.ShapeDtypeStruct((NW, 16), jnp.float32),
        scratch_types=[
            pltpu.VMEM((b_per_w,), jnp.int32),
            pltpu.VMEM((b_per_w, 8, 128), jnp.float32),
            pltpu.VMEM((b_per_w // 8, 8, 128), jnp.float32),
            pltpu.VMEM((16,), jnp.float32),
            pltpu.SemaphoreType.DMA,
        ],
    )(functools.partial(_sc_gather_body, V=V, b_per_w=b_per_w,
                        n_sub=n_sub, NC=NC))
    sc_part = sc_fn(pred, gold)

    s_all = dense[0, 0] + jnp.sum(sc_dense)
    return B * k1 - s * s_all + jnp.sum(sc_part)
